# baseline (speedup 1.0000x reference)
import math

import jax
import jax.numpy as jnp
from jax import lax
from jax.experimental import pallas as pl
from jax.experimental.pallas import tpu as pltpu
from jax.experimental.pallas import tpu_sc as plsc

N = 10000
E = 320000
K = 10
ALPHA = 0.1
S = 4.0
M = 0.5

NTILES = 16
EW = E // NTILES
ROWS = 16
RL = 1024
NPAD = ROWS * RL
GROUPS = EW // 16


def _rsqrt16(x):
    i = plsc.bitcast(x, jnp.int32)
    i = jnp.int32(0x5F3759DF) - (i >> 1)
    y = plsc.bitcast(i, jnp.float32)
    half = x * 0.5
    for _ in range(3):
        y = y * (1.5 - half * y * y)
    return y


def _sc_weights_body(ei_hbm, rows10_hbm, out_hbm, src_v, dst_v, u2d,
                     acc2d, norm2d, wacc_v, nsave_v, idx10_v, sem_a, sem_b,
                     shared):
    wid = lax.axis_index("s")
    ones16 = jnp.full((16,), 1.0, jnp.float32)
    zeros16 = jnp.zeros((16,), jnp.float32)

    cp_a = pltpu.make_async_copy(ei_hbm.at[0, pl.ds(wid * EW, EW)], src_v,
                                 sem_a)
    cp_a.start()
    cp_b = pltpu.make_async_copy(ei_hbm.at[1, pl.ds(wid * EW, EW)], dst_v,
                                 sem_b)
    cp_b.start()
    pltpu.sync_copy(rows10_hbm, idx10_v)
    cp_b.wait()

    NR = (N + RL - 1) // RL

    def _clear_acc():
        @plsc.parallel_loop(0, NR * (RL // 16), unroll=8)
        def _z(g):
            acc2d[g >> 6, pl.ds((g & 63) * 16, 16)] = zeros16

    def _reduce_to_u(post=None):
        @pl.when(wid == 0)
        def _():
            pltpu.sync_copy(acc2d.at[pl.ds(0, NR)], shared.at[pl.ds(0, NR)])
        plsc.subcore_barrier()

        @pl.when(wid != 0)
        def _():
            pltpu.sync_copy(acc2d.at[pl.ds(0, NR)], shared.at[idx10_v],
                            add=True)
        _clear_acc()
        plsc.subcore_barrier()
        pltpu.sync_copy(shared.at[pl.ds(0, NR)], u2d.at[pl.ds(0, NR)])
        if post is not None:
            post()
        plsc.subcore_barrier()

    if True:
        _clear_acc()

        @plsc.parallel_loop(0, GROUPS, unroll=8)
        def _deg(i):
            d = dst_v[pl.ds(i * 16, 16)]
            plsc.addupdate_scatter(acc2d, [d >> 10, d & 1023], ones16)

        def _post_deg():
            @plsc.parallel_loop(0, NR * (RL // 16), unroll=4)
            def _norm(g):
                r = g >> 6
                sl = pl.ds((g & 63) * 16, 16)
                x = jnp.maximum(u2d[r, sl], 1.0)
                y = _rsqrt16(x)
                u2d[r, sl] = y
                norm2d[r, sl] = y * y

            c0 = ALPHA / N
            @plsc.parallel_loop(0, RL // 16, unroll=4)
            def _nsave(c):
                sl = pl.ds(c * 16, 16)
                nv = u2d[wid, sl]
                nsave_v[sl] = nv
                wacc_v[sl] = c0 * nv

            cp_a.wait()

            @plsc.parallel_loop(0, GROUPS, unroll=8)
            def _pack(i):
                sl = pl.ds(i * 16, 16)
                dst_v[sl] = (src_v[sl] << 14) | dst_v[sl]

        _reduce_to_u(_post_deg)

        for j in range(1, K + 1):
            @plsc.parallel_loop(0, GROUPS, unroll=16)
            def _prop(i):
                sl = pl.ds(i * 16, 16)
                p = dst_v[sl]
                vals = plsc.load_gather(u2d, [(p >> 10) & 15, p & 1023])
                plsc.addupdate_scatter(acc2d, [p >> 24, (p >> 14) & 1023],
                                       vals)

            coef = (1.0 - ALPHA) ** j * (ALPHA if j < K else 1.0) / N
            if j < K:
                def _post_round(coef=coef):
                    @plsc.parallel_loop(0, NR * (RL // 16), unroll=4)
                    def _scale(g):
                        r = g >> 6
                        sl = pl.ds((g & 63) * 16, 16)
                        u2d[r, sl] = u2d[r, sl] * norm2d[r, sl]

                    @plsc.parallel_loop(0, RL // 16, unroll=4)
                    def _wadd(c):
                        sl = pl.ds(c * 16, 16)
                        wacc_v[sl] = wacc_v[sl] + coef * u2d[wid, sl]
            else:
                def _post_round(coef=coef):
                    @plsc.parallel_loop(0, RL // 16, unroll=4)
                    def _wlast(c):
                        sl = pl.ds(c * 16, 16)
                        wacc_v[sl] = wacc_v[sl] + coef * (
                            u2d[wid, sl] * norm2d[wid, sl])
            _reduce_to_u(_post_round)

        @plsc.parallel_loop(0, RL // 16, unroll=4)
        def _wfin(c):
            sl = pl.ds(c * 16, 16)
            wacc_v[sl] = wacc_v[sl] / nsave_v[sl]

        pltpu.sync_copy(wacc_v, out_hbm.at[pl.ds(wid * RL, RL)])


@jax.jit
def _sc_weights(edge_index):
    mesh = plsc.VectorSubcoreMesh(core_axis_name="c", subcore_axis_name="s",
                                  num_cores=1, num_subcores=NTILES)
    rows10 = jnp.arange(10, dtype=jnp.int32)
    return pl.kernel(
        _sc_weights_body,
        out_type=jax.ShapeDtypeStruct((NPAD,), jnp.float32),
        mesh=mesh,
        compiler_params=pltpu.CompilerParams(use_tc_tiling_on_sc=False,
                                             needs_layout_passes=False),
        scratch_types=[
            pltpu.VMEM((EW,), jnp.int32),
            pltpu.VMEM((EW,), jnp.int32),
            pltpu.VMEM((ROWS, RL), jnp.float32),
            pltpu.VMEM((ROWS, RL), jnp.float32),
            pltpu.VMEM((ROWS, RL), jnp.float32),
            pltpu.VMEM((RL,), jnp.float32),
            pltpu.VMEM((RL,), jnp.float32),
            pltpu.VMEM((10,), jnp.int32),
            pltpu.SemaphoreType.DMA,
            pltpu.SemaphoreType.DMA,
            pltpu.VMEM_SHARED((ROWS, RL), jnp.float32),
        ],
    )(edge_index, rows10)


def _tc_body(x_ref, w1_ref, b1_ref, w_ref, w2_ref, b2_ref, aw_ref, lab_ref,
             o_ref):
    z = jnp.maximum(
        jnp.dot(x_ref[...], w1_ref[...], preferred_element_type=jnp.float32)
        + b1_ref[...], 0.0)
    wrow = w_ref[...]
    u = jnp.dot(wrow, z, preferred_element_type=jnp.float32)
    sw = jnp.sum(wrow)
    emb = jnp.dot(u, w2_ref[...], preferred_element_type=jnp.float32) \
        + sw * b2_ref[...]
    en = emb * lax.rsqrt(jnp.sum(emb * emb))
    aw = aw_ref[...]
    wn = aw * lax.rsqrt(jnp.sum(aw * aw, axis=1, keepdims=True))
    cos1 = jnp.sum(wn * en, axis=1, keepdims=True)
    cos1 = jnp.clip(cos1, -1.0 + 1e-7, 1.0 - 1e-7)
    rows = lax.broadcasted_iota(jnp.int32, (512, 1), 0)
    hit = rows == lab_ref[0]
    marg = cos1 * math.cos(M) - jnp.sqrt(1.0 - cos1 * cos1) * math.sin(M)
    o_ref[...] = jnp.where(hit, marg, cos1) * S


@jax.jit
def _tc_all(features, W1, b1, w_row, W2, b2, arc_w, labels):
    return pl.pallas_call(
        _tc_body,
        out_shape=jax.ShapeDtypeStruct((512, 1), jnp.float32),
        in_specs=[
            pl.BlockSpec(memory_space=pltpu.VMEM),
            pl.BlockSpec(memory_space=pltpu.VMEM),
            pl.BlockSpec(memory_space=pltpu.VMEM),
            pl.BlockSpec(memory_space=pltpu.VMEM),
            pl.BlockSpec(memory_space=pltpu.VMEM),
            pl.BlockSpec(memory_space=pltpu.VMEM),
            pl.BlockSpec(memory_space=pltpu.VMEM),
            pl.BlockSpec(memory_space=pltpu.SMEM),
        ],
    )(features, W1, b1.reshape(1, 128), w_row, W2, b2.reshape(1, 128),
      arc_w, labels)


def kernel(features, edge_index, labels, W1, b1, W2, b2, arc_w):
    w_pad = _sc_weights(edge_index)
    w_row = w_pad[:N].reshape(1, N)
    out = _tc_all(features, W1, b1, w_row, W2, b2, arc_w, labels)
    return out.reshape(1, 512)


# speedup vs baseline: 1.0112x; 1.0112x over previous
import math

import jax
import jax.numpy as jnp
from jax import lax
from jax.experimental import pallas as pl
from jax.experimental.pallas import tpu as pltpu
from jax.experimental.pallas import tpu_sc as plsc

N = 10000
E = 320000
K = 10
ALPHA = 0.1
S = 4.0
M = 0.5

NTILES = 16
EW = E // NTILES
ROWS = 16
RL = 1024
NPAD = ROWS * RL
GROUPS = EW // 16


def _rsqrt16(x):
    i = plsc.bitcast(x, jnp.int32)
    i = jnp.int32(0x5F3759DF) - (i >> 1)
    y = plsc.bitcast(i, jnp.float32)
    half = x * 0.5
    for _ in range(3):
        y = y * (1.5 - half * y * y)
    return y


def _sc_weights_body(ei_hbm, rows10_hbm, out_hbm, src_v, dst_v, u2d,
                     acc2d, norm2d, wacc_v, nsave_v, idx10_v, sem_a, sem_b,
                     shared):
    wid = lax.axis_index("s")
    ones16 = jnp.full((16,), 1.0, jnp.float32)
    zeros16 = jnp.zeros((16,), jnp.float32)

    cp_a = pltpu.make_async_copy(ei_hbm.at[0, pl.ds(wid * EW, EW)], src_v,
                                 sem_a)
    cp_a.start()
    cp_b = pltpu.make_async_copy(ei_hbm.at[1, pl.ds(wid * EW, EW)], dst_v,
                                 sem_b)
    cp_b.start()
    pltpu.sync_copy(rows10_hbm, idx10_v)
    cp_b.wait()

    NR = (N + RL - 1) // RL

    def _clear_acc():
        @plsc.parallel_loop(0, NR * (RL // 16), unroll=8)
        def _z(g):
            acc2d[g >> 6, pl.ds((g & 63) * 16, 16)] = zeros16

    def _reduce_to_u(post=None):
        @pl.when(wid == 0)
        def _():
            pltpu.sync_copy(acc2d.at[pl.ds(0, NR)], shared.at[pl.ds(0, NR)])
        plsc.subcore_barrier()

        @pl.when(wid != 0)
        def _():
            pltpu.sync_copy(acc2d.at[pl.ds(0, NR)], shared.at[idx10_v],
                            add=True)
        _clear_acc()
        plsc.subcore_barrier()
        pltpu.sync_copy(shared.at[pl.ds(0, NR)], u2d.at[pl.ds(0, NR)])
        if post is not None:
            post()
        plsc.subcore_barrier()

    if True:
        _clear_acc()

        @plsc.parallel_loop(0, GROUPS, unroll=8)
        def _deg(i):
            d = dst_v[pl.ds(i * 16, 16)]
            plsc.addupdate_scatter(acc2d, [d >> 10, d & 1023], ones16)

        def _post_deg():
            @plsc.parallel_loop(0, NR * (RL // 16), unroll=4)
            def _norm(g):
                r = g >> 6
                sl = pl.ds((g & 63) * 16, 16)
                x = jnp.maximum(u2d[r, sl], 1.0)
                y = _rsqrt16(x)
                u2d[r, sl] = y
                norm2d[r, sl] = y * y

            c0 = ALPHA / N
            @plsc.parallel_loop(0, RL // 16, unroll=4)
            def _nsave(c):
                sl = pl.ds(c * 16, 16)
                nv = u2d[wid, sl]
                nsave_v[sl] = nv
                wacc_v[sl] = c0 * nv

            cp_a.wait()

            @plsc.parallel_loop(0, GROUPS, unroll=8)
            def _pack(i):
                sl = pl.ds(i * 16, 16)
                dst_v[sl] = (src_v[sl] << 14) | dst_v[sl]

        _reduce_to_u(_post_deg)

        for j in range(1, K + 1):
            @plsc.parallel_loop(0, GROUPS, unroll=32)
            def _prop(i):
                sl = pl.ds(i * 16, 16)
                p = dst_v[sl]
                vals = plsc.load_gather(u2d, [(p >> 10) & 15, p & 1023])
                plsc.addupdate_scatter(acc2d, [p >> 24, (p >> 14) & 1023],
                                       vals)

            coef = (1.0 - ALPHA) ** j * (ALPHA if j < K else 1.0) / N
            if j < K:
                def _post_round(coef=coef):
                    @plsc.parallel_loop(0, NR * (RL // 16), unroll=4)
                    def _scale(g):
                        r = g >> 6
                        sl = pl.ds((g & 63) * 16, 16)
                        u2d[r, sl] = u2d[r, sl] * norm2d[r, sl]

                    @plsc.parallel_loop(0, RL // 16, unroll=4)
                    def _wadd(c):
                        sl = pl.ds(c * 16, 16)
                        wacc_v[sl] = wacc_v[sl] + coef * u2d[wid, sl]
            else:
                def _post_round(coef=coef):
                    @plsc.parallel_loop(0, RL // 16, unroll=4)
                    def _wlast(c):
                        sl = pl.ds(c * 16, 16)
                        wacc_v[sl] = wacc_v[sl] + coef * (
                            u2d[wid, sl] * norm2d[wid, sl])
            _reduce_to_u(_post_round)

        @plsc.parallel_loop(0, RL // 16, unroll=4)
        def _wfin(c):
            sl = pl.ds(c * 16, 16)
            wacc_v[sl] = wacc_v[sl] / nsave_v[sl]

        pltpu.sync_copy(wacc_v, out_hbm.at[pl.ds(wid * RL, RL)])


@jax.jit
def _sc_weights(edge_index):
    mesh = plsc.VectorSubcoreMesh(core_axis_name="c", subcore_axis_name="s",
                                  num_cores=1, num_subcores=NTILES)
    rows10 = jnp.arange(10, dtype=jnp.int32)
    return pl.kernel(
        _sc_weights_body,
        out_type=jax.ShapeDtypeStruct((NPAD,), jnp.float32),
        mesh=mesh,
        compiler_params=pltpu.CompilerParams(use_tc_tiling_on_sc=False,
                                             needs_layout_passes=False),
        scratch_types=[
            pltpu.VMEM((EW,), jnp.int32),
            pltpu.VMEM((EW,), jnp.int32),
            pltpu.VMEM((ROWS, RL), jnp.float32),
            pltpu.VMEM((ROWS, RL), jnp.float32),
            pltpu.VMEM((ROWS, RL), jnp.float32),
            pltpu.VMEM((RL,), jnp.float32),
            pltpu.VMEM((RL,), jnp.float32),
            pltpu.VMEM((10,), jnp.int32),
            pltpu.SemaphoreType.DMA,
            pltpu.SemaphoreType.DMA,
            pltpu.VMEM_SHARED((ROWS, RL), jnp.float32),
        ],
    )(edge_index, rows10)


def _tc_body(x_ref, w1_ref, b1_ref, w_ref, w2_ref, b2_ref, aw_ref, lab_ref,
             o_ref):
    z = jnp.maximum(
        jnp.dot(x_ref[...], w1_ref[...], preferred_element_type=jnp.float32)
        + b1_ref[...], 0.0)
    wrow = w_ref[...]
    u = jnp.dot(wrow, z, preferred_element_type=jnp.float32)
    sw = jnp.sum(wrow)
    emb = jnp.dot(u, w2_ref[...], preferred_element_type=jnp.float32) \
        + sw * b2_ref[...]
    en = emb * lax.rsqrt(jnp.sum(emb * emb))
    aw = aw_ref[...]
    wn = aw * lax.rsqrt(jnp.sum(aw * aw, axis=1, keepdims=True))
    cos1 = jnp.sum(wn * en, axis=1, keepdims=True)
    cos1 = jnp.clip(cos1, -1.0 + 1e-7, 1.0 - 1e-7)
    rows = lax.broadcasted_iota(jnp.int32, (512, 1), 0)
    hit = rows == lab_ref[0]
    marg = cos1 * math.cos(M) - jnp.sqrt(1.0 - cos1 * cos1) * math.sin(M)
    o_ref[...] = jnp.where(hit, marg, cos1) * S


@jax.jit
def _tc_all(features, W1, b1, w_row, W2, b2, arc_w, labels):
    return pl.pallas_call(
        _tc_body,
        out_shape=jax.ShapeDtypeStruct((512, 1), jnp.float32),
        in_specs=[
            pl.BlockSpec(memory_space=pltpu.VMEM),
            pl.BlockSpec(memory_space=pltpu.VMEM),
            pl.BlockSpec(memory_space=pltpu.VMEM),
            pl.BlockSpec(memory_space=pltpu.VMEM),
            pl.BlockSpec(memory_space=pltpu.VMEM),
            pl.BlockSpec(memory_space=pltpu.VMEM),
            pl.BlockSpec(memory_space=pltpu.VMEM),
            pl.BlockSpec(memory_space=pltpu.SMEM),
        ],
    )(features, W1, b1.reshape(1, 128), w_row, W2, b2.reshape(1, 128),
      arc_w, labels)


def kernel(features, edge_index, labels, W1, b1, W2, b2, arc_w):
    w_pad = _sc_weights(edge_index)
    w_row = w_pad[:N].reshape(1, N)
    out = _tc_all(features, W1, b1, w_row, W2, b2, arc_w, labels)
    return out.reshape(1, 512)
